# Initial kernel scaffold; baseline (speedup 1.0000x reference)
import jax
import jax.numpy as jnp
from jax.experimental import pallas as pl


def kernel(X):
    raise NotImplementedError("write your pallas kernel here")


# speedup vs baseline: 1.6976x; 1.6976x over previous
import jax
import jax.numpy as jnp
from jax.experimental import pallas as pl
from jax.experimental.pallas import tpu as pltpu

K = 512
ITERS = 4
D = 64
DP = 128
B = 2048
N = 32768

_PREC = jax.lax.Precision.HIGHEST


def _body(xa_ref, c0_ref, out_ref, cent_ref, sums_ref):
    it = pl.program_id(0)
    b = pl.program_id(1)

    @pl.when(b == 0)
    def _new_iter():
        @pl.when(it == 0)
        def _():
            cent_ref[...] = c0_ref[...]

        @pl.when(it > 0)
        def _():
            s = sums_ref[...]
            cnt = s[:, D:D + 1]
            lane = jax.lax.broadcasted_iota(jnp.int32, (K, DP), 1)
            cent_ref[...] = jnp.where(lane < D, s / cnt, 0.0)

        sums_ref[...] = jnp.zeros_like(sums_ref)

    xa = xa_ref[...]
    ca = cent_ref[...]
    xs = xa[:, :D]
    cs = ca[:, :D]
    x2 = jnp.sum(xs * xs, axis=1, keepdims=True)
    c2 = jnp.sum(cs * cs, axis=1)
    xc = jax.lax.dot_general(xs, cs, (((1,), (1,)), ((), ())),
                             precision=jax.lax.Precision.DEFAULT)
    d2 = jnp.maximum(x2 + c2[None, :] - 2.0 * xc, 0.0)
    lbl = jnp.argmin(d2, axis=1).astype(jnp.int32)

    @pl.when(it == ITERS - 1)
    def _():
        out_ref[...] = lbl

    @pl.when(it < ITERS - 1)
    def _accumulate():
        col = jax.lax.broadcasted_iota(jnp.int32, (B, K), 1)
        oh = (col == lbl[:, None]).astype(jnp.float32)
        sums_ref[...] += jax.lax.dot_general(
            oh, xa, (((0,), (0,)), ((), ())), precision=_PREC)


def kernel(X):
    n, d = X.shape
    ones = jnp.ones((n, 1), X.dtype)
    zpad = jnp.zeros((n, DP - D - 1), X.dtype)
    xa = jnp.concatenate([X, ones, zpad], axis=1)
    c0 = jnp.concatenate([X[:K], jnp.zeros((K, DP - D), X.dtype)], axis=1)

    return pl.pallas_call(
        _body,
        grid=(ITERS, N // B),
        in_specs=[
            pl.BlockSpec((B, DP), lambda it, b: (b, 0)),
            pl.BlockSpec((K, DP), lambda it, b: (0, 0)),
        ],
        out_specs=pl.BlockSpec((B,), lambda it, b: (b,)),
        out_shape=jax.ShapeDtypeStruct((N,), jnp.int32),
        scratch_shapes=[
            pltpu.VMEM((K, DP), jnp.float32),
            pltpu.VMEM((K, DP), jnp.float32),
        ],
        compiler_params=pltpu.CompilerParams(
            dimension_semantics=("arbitrary", "arbitrary"),
        ),
    )(xa, c0)
